# Initial kernel scaffold; baseline (speedup 1.0000x reference)
import jax
import jax.numpy as jnp
from jax.experimental import pallas as pl


def kernel(hidden_states, tgt_route, W_gate, Wg, Wu, Wd, Ws_g, Ws_u, Ws_d):
    raise NotImplementedError("write your pallas kernel here")


# speedup vs baseline: 2.1799x; 2.1799x over previous
import jax
import jax.numpy as jnp
from jax.experimental import pallas as pl
from jax.experimental.pallas import tpu as pltpu

_E, _K, _H, _M, _SH = 64, 6, 128, 80, 160
_T = 256


def _moe_block_kernel(x_ref, r_ref, wg_ref, wgt_ref, wut_ref, wdt_ref,
                      rmap_ref, wsg_ref, wsu_ref, wsd_ref, y_ref, scores_ref):
    x = x_ref[...]
    r = r_ref[...]

    logits = jnp.dot(r, wg_ref[...], preferred_element_type=jnp.float32)
    mx = jnp.max(logits, axis=1, keepdims=True)
    ex = jnp.exp(logits - mx)
    scores = ex / jnp.sum(ex, axis=1, keepdims=True)
    scores_ref[...] = scores

    iota = jax.lax.broadcasted_iota(jnp.int32, scores.shape, 1)
    remaining = scores
    sel = jnp.zeros(scores.shape, jnp.bool_)
    for _ in range(_K):
        m = jnp.max(remaining, axis=1, keepdims=True)
        first = jnp.min(jnp.where(remaining == m, iota, _E), axis=1,
                        keepdims=True)
        pick = iota == first
        sel = jnp.logical_or(sel, pick)
        remaining = jnp.where(pick, -jnp.inf, remaining)
    wts = jnp.where(sel, scores, 0.0)
    wts = wts / (jnp.sum(wts, axis=1, keepdims=True) + 1e-20)

    xb = x.astype(jnp.bfloat16)
    h1 = jnp.dot(xb, wgt_ref[...], preferred_element_type=jnp.float32)
    h2 = jnp.dot(xb, wut_ref[...], preferred_element_type=jnp.float32)
    act = jax.nn.silu(h1) * h2
    wwide = jnp.dot(wts, rmap_ref[...], preferred_element_type=jnp.float32)
    scaled = (act * wwide).astype(jnp.bfloat16)
    y = jnp.dot(scaled, wdt_ref[...], preferred_element_type=jnp.float32)

    sg = jnp.dot(xb, wsg_ref[...], preferred_element_type=jnp.float32)
    su = jnp.dot(xb, wsu_ref[...], preferred_element_type=jnp.float32)
    sact = (jax.nn.silu(sg) * su).astype(jnp.bfloat16)
    y = y + jnp.dot(sact, wsd_ref[...], preferred_element_type=jnp.float32)

    y_ref[...] = y + x


def kernel(hidden_states, tgt_route, W_gate, Wg, Wu, Wd, Ws_g, Ws_u, Ws_d):
    B, S, H = hidden_states.shape
    N = B * S
    x = hidden_states.reshape(N, H)
    r = tgt_route.reshape(N, H)

    wgT = W_gate.T
    wgtT = Wg.transpose(2, 0, 1).reshape(H, _E * _M).astype(jnp.bfloat16)
    wutT = Wu.transpose(2, 0, 1).reshape(H, _E * _M).astype(jnp.bfloat16)
    wdtT = Wd.transpose(0, 2, 1).reshape(_E * _M, H).astype(jnp.bfloat16)
    rmap = (jnp.arange(_E)[:, None] == (jnp.arange(_E * _M)[None, :] // _M)
            ).astype(jnp.float32)
    wsgT = Ws_g.T.astype(jnp.bfloat16)
    wsuT = Ws_u.T.astype(jnp.bfloat16)
    wsdT = Ws_d.T.astype(jnp.bfloat16)

    grid = (N // _T,)
    tok = lambda i: (i, 0)
    full = lambda i: (0, 0)
    y, scores = pl.pallas_call(
        _moe_block_kernel,
        grid=grid,
        in_specs=[
            pl.BlockSpec((_T, H), tok),
            pl.BlockSpec((_T, H), tok),
            pl.BlockSpec((H, _E), full),
            pl.BlockSpec((H, _E * _M), full),
            pl.BlockSpec((H, _E * _M), full),
            pl.BlockSpec((_E * _M, H), full),
            pl.BlockSpec((_E, _E * _M), full),
            pl.BlockSpec((H, _SH), full),
            pl.BlockSpec((H, _SH), full),
            pl.BlockSpec((_SH, H), full),
        ],
        out_specs=[
            pl.BlockSpec((_T, H), tok),
            pl.BlockSpec((_T, _E), tok),
        ],
        out_shape=[
            jax.ShapeDtypeStruct((N, H), jnp.float32),
            jax.ShapeDtypeStruct((N, _E), jnp.float32),
        ],
        compiler_params=pltpu.CompilerParams(
            dimension_semantics=("parallel",)),
    )(x, r, wgT, wgtT, wutT, wdtT, rmap, wsgT, wsuT, wsdT)
    return y.reshape(B, S, H), scores


# speedup vs baseline: 2.1851x; 1.0024x over previous
import jax
import jax.numpy as jnp
from jax.experimental import pallas as pl
from jax.experimental.pallas import tpu as pltpu

_E, _K, _H, _M, _SH = 64, 6, 128, 80, 160
_T = 256


def _moe_block_kernel(x_ref, r_ref, wg_ref, wgt_ref, wdt_ref,
                      rmap_ref, wsg_ref, wsu_ref, wsd_ref, y_ref, scores_ref):
    x = x_ref[...]
    r = r_ref[...]

    logits = jnp.dot(r, wg_ref[...], preferred_element_type=jnp.float32)
    mx = jnp.max(logits, axis=1, keepdims=True)
    ex = jnp.exp(logits - mx)
    scores = ex / jnp.sum(ex, axis=1, keepdims=True)
    scores_ref[...] = scores

    iota = jax.lax.broadcasted_iota(jnp.int32, scores.shape, 1)
    remaining = scores
    sel = jnp.zeros(scores.shape, jnp.bool_)
    for _ in range(_K):
        m = jnp.max(remaining, axis=1, keepdims=True)
        first = jnp.min(jnp.where(remaining == m, iota, _E), axis=1,
                        keepdims=True)
        pick = iota == first
        sel = jnp.logical_or(sel, pick)
        remaining = jnp.where(pick, -jnp.inf, remaining)
    wts = jnp.where(sel, scores, 0.0)
    wts = wts / (jnp.sum(wts, axis=1, keepdims=True) + 1e-20)

    xb = x.astype(jnp.bfloat16)
    h = jnp.dot(xb, wgt_ref[...], preferred_element_type=jnp.float32)
    h1 = h[:, :_E * _M]
    h2 = h[:, _E * _M:]
    act = jax.nn.silu(h1) * h2
    wwide = jnp.dot(wts.astype(jnp.bfloat16), rmap_ref[...],
                    preferred_element_type=jnp.float32)
    scaled = (act * wwide).astype(jnp.bfloat16)
    y = jnp.dot(scaled, wdt_ref[...], preferred_element_type=jnp.float32)

    sg = jnp.dot(xb, wsg_ref[...], preferred_element_type=jnp.float32)
    su = jnp.dot(xb, wsu_ref[...], preferred_element_type=jnp.float32)
    sact = (jax.nn.silu(sg) * su).astype(jnp.bfloat16)
    y = y + jnp.dot(sact, wsd_ref[...], preferred_element_type=jnp.float32)

    y_ref[...] = y + x


def kernel(hidden_states, tgt_route, W_gate, Wg, Wu, Wd, Ws_g, Ws_u, Ws_d):
    B, S, H = hidden_states.shape
    N = B * S
    x = hidden_states.reshape(N, H)
    r = tgt_route.reshape(N, H)

    wgT = W_gate.T
    wgtT = Wg.transpose(2, 0, 1).reshape(H, _E * _M).astype(jnp.bfloat16)
    wutT = Wu.transpose(2, 0, 1).reshape(H, _E * _M).astype(jnp.bfloat16)
    wguT = jnp.concatenate([wgtT, wutT], axis=1)
    wdtT = Wd.transpose(0, 2, 1).reshape(_E * _M, H).astype(jnp.bfloat16)
    rmap = (jnp.arange(_E)[:, None] == (jnp.arange(_E * _M)[None, :] // _M)
            ).astype(jnp.bfloat16)
    wsgT = Ws_g.T.astype(jnp.bfloat16)
    wsuT = Ws_u.T.astype(jnp.bfloat16)
    wsdT = Ws_d.T.astype(jnp.bfloat16)

    grid = (N // _T,)
    tok = lambda i: (i, 0)
    full = lambda i: (0, 0)
    y, scores = pl.pallas_call(
        _moe_block_kernel,
        grid=grid,
        in_specs=[
            pl.BlockSpec((_T, H), tok),
            pl.BlockSpec((_T, H), tok),
            pl.BlockSpec((H, _E), full),
            pl.BlockSpec((H, 2 * _E * _M), full),
            pl.BlockSpec((_E * _M, H), full),
            pl.BlockSpec((_E, _E * _M), full),
            pl.BlockSpec((H, _SH), full),
            pl.BlockSpec((H, _SH), full),
            pl.BlockSpec((_SH, H), full),
        ],
        out_specs=[
            pl.BlockSpec((_T, H), tok),
            pl.BlockSpec((_T, _E), tok),
        ],
        out_shape=[
            jax.ShapeDtypeStruct((N, H), jnp.float32),
            jax.ShapeDtypeStruct((N, _E), jnp.float32),
        ],
        compiler_params=pltpu.CompilerParams(
            dimension_semantics=("parallel",)),
    )(x, r, wgT, wguT, wdtT, rmap, wsgT, wsuT, wsdT)
    return y.reshape(B, S, H), scores


# speedup vs baseline: 2.3332x; 1.0678x over previous
import jax
import jax.numpy as jnp
from jax.experimental import pallas as pl
from jax.experimental.pallas import tpu as pltpu

_E, _K, _H, _M, _SH = 64, 6, 128, 80, 160
_T = 256


def _moe_block_kernel(x_ref, r_ref, wg_ref, wgt_ref, wdt_ref,
                      rmap_ref, wsg_ref, wsd_ref, y_ref, scores_ref):
    x = x_ref[...]
    r = r_ref[...]

    logits = jnp.dot(r, wg_ref[...], preferred_element_type=jnp.float32)
    mx = jnp.max(logits, axis=1, keepdims=True)
    ex = jnp.exp(logits - mx)
    scores = ex / jnp.sum(ex, axis=1, keepdims=True)
    scores_ref[...] = scores

    iota = jax.lax.broadcasted_iota(jnp.int32, scores.shape, 1)
    sbits = jax.lax.bitcast_convert_type(scores, jnp.int32)
    key = jax.lax.bitwise_or(jax.lax.bitwise_and(sbits, ~jnp.int32(_E - 1)),
                             (_E - 1) - iota)
    sel = jnp.zeros(scores.shape, jnp.bool_)
    for _ in range(_K):
        m = jnp.max(key, axis=1, keepdims=True)
        pick = key == m
        sel = jnp.logical_or(sel, pick)
        key = jnp.where(pick, jnp.int32(-1), key)
    wts = jnp.where(sel, scores, 0.0)
    wts = wts / (jnp.sum(wts, axis=1, keepdims=True) + 1e-20)

    xb = x.astype(jnp.bfloat16)
    h = jnp.dot(xb, wgt_ref[...], preferred_element_type=jnp.float32)
    h1 = h[:, :_E * _M]
    h2 = h[:, _E * _M:]
    act = (0.5 * h1 + 0.5 * h1 * jnp.tanh(0.5 * h1)) * h2
    wwide = jnp.dot(wts.astype(jnp.bfloat16), rmap_ref[...],
                    preferred_element_type=jnp.float32)
    scaled = (act * wwide).astype(jnp.bfloat16)
    y = jnp.dot(scaled, wdt_ref[...], preferred_element_type=jnp.float32)

    sh = jnp.dot(xb, wsg_ref[...], preferred_element_type=jnp.float32)
    sg = sh[:, :_SH]
    su = sh[:, _SH:]
    sact = ((0.5 * sg + 0.5 * sg * jnp.tanh(0.5 * sg)) * su).astype(jnp.bfloat16)
    y = y + jnp.dot(sact, wsd_ref[...], preferred_element_type=jnp.float32)

    y_ref[...] = y + x


def kernel(hidden_states, tgt_route, W_gate, Wg, Wu, Wd, Ws_g, Ws_u, Ws_d):
    B, S, H = hidden_states.shape
    N = B * S
    x = hidden_states.reshape(N, H)
    r = tgt_route.reshape(N, H)

    wgT = W_gate.T
    wgtT = Wg.transpose(2, 0, 1).reshape(H, _E * _M).astype(jnp.bfloat16)
    wutT = Wu.transpose(2, 0, 1).reshape(H, _E * _M).astype(jnp.bfloat16)
    wguT = jnp.concatenate([wgtT, wutT], axis=1)
    wdtT = Wd.transpose(0, 2, 1).reshape(_E * _M, H).astype(jnp.bfloat16)
    rmap = (jnp.arange(_E)[:, None] == (jnp.arange(_E * _M)[None, :] // _M)
            ).astype(jnp.bfloat16)
    wsguT = jnp.concatenate([Ws_g.T, Ws_u.T], axis=1).astype(jnp.bfloat16)
    wsdT = Ws_d.T.astype(jnp.bfloat16)

    grid = (N // _T,)
    tok = lambda i: (i, 0)
    full = lambda i: (0, 0)
    y, scores = pl.pallas_call(
        _moe_block_kernel,
        grid=grid,
        in_specs=[
            pl.BlockSpec((_T, H), tok),
            pl.BlockSpec((_T, H), tok),
            pl.BlockSpec((H, _E), full),
            pl.BlockSpec((H, 2 * _E * _M), full),
            pl.BlockSpec((_E * _M, H), full),
            pl.BlockSpec((_E, _E * _M), full),
            pl.BlockSpec((H, 2 * _SH), full),
            pl.BlockSpec((_SH, H), full),
        ],
        out_specs=[
            pl.BlockSpec((_T, H), tok),
            pl.BlockSpec((_T, _E), tok),
        ],
        out_shape=[
            jax.ShapeDtypeStruct((N, H), jnp.float32),
            jax.ShapeDtypeStruct((N, _E), jnp.float32),
        ],
        compiler_params=pltpu.CompilerParams(
            dimension_semantics=("parallel",)),
    )(x, r, wgT, wguT, wdtT, rmap, wsguT, wsdT)
    return y.reshape(B, S, H), scores
